# Initial kernel scaffold; baseline (speedup 1.0000x reference)
import jax
import jax.numpy as jnp
from jax.experimental import pallas as pl


def kernel(x, x_t_slot, y, y_t_slot, vecs_use, time_embeddings, W_out1, b_out1, W_out2, b_out2, W_in1, b_in1, W_in2, b_in2):
    raise NotImplementedError("write your pallas kernel here")


# speedup vs baseline: 5.1731x; 5.1731x over previous
import functools

import jax
import jax.numpy as jnp
from jax import lax
from jax.experimental import pallas as pl
from jax.experimental.pallas import tpu as pltpu
from jax.experimental.pallas import tpu_sc as plsc

_NC = 2
_NS = 16
_NW = _NC * _NS


def _make_sc_gather(V, D, B, C):
    n_per_w = B // _NW
    n_iter = n_per_w // C
    assert n_per_w % C == 0 and C % 8 == 0

    mesh = plsc.VectorSubcoreMesh(core_axis_name="c", subcore_axis_name="s")

    @functools.partial(
        pl.kernel,
        mesh=mesh,
        out_type=jax.ShapeDtypeStruct((B, D), jnp.float32),
        scratch_types=[
            pltpu.VMEM((C,), jnp.int32),
            pltpu.VMEM((C, D), jnp.float32),
            pltpu.SemaphoreType.DMA,
        ],
        compiler_params=pltpu.CompilerParams(use_tc_tiling_on_sc=False),
    )
    def gather(table_hbm, idx_hbm, out_hbm, idx_v, rows_v, sem):
        wid = lax.axis_index("s") * _NC + lax.axis_index("c")
        for i in range(n_iter):
            base = wid * n_per_w + i * C
            pltpu.sync_copy(idx_hbm.at[pl.ds(base, C)], idx_v)
            pltpu.async_copy(table_hbm.at[idx_v], rows_v, sem).wait()
            pltpu.sync_copy(rows_v, out_hbm.at[pl.ds(base, C)])

    return gather


def _segment(t):
    hd = t % 24
    return jnp.where((hd >= 22) | (hd < 6), 0, jnp.where(hd < 14, 1, 2))


def _mlp_loss_body(n_total, g_ref, xt_ref, yt_ref, te_ref, wo1_ref, bo1_ref,
                   wo2_ref, bo2_ref, wi1_ref, bi1_ref, wi2_ref, bi2_ref,
                   out_ref):
    i = pl.program_id(0)
    nb = pl.num_programs(0)
    D = te_ref.shape[1]

    xg = g_ref[0]
    yg = g_ref[1]
    ng = g_ref[2]
    te = te_ref[...]
    wo1a = wo1_ref[:D, :]
    wo1b = wo1_ref[D:, :]
    wi1a = wi1_ref[:D, :]
    wi1b = wi1_ref[D:, :]

    tb_out = jnp.dot(te, wo1b, preferred_element_type=jnp.float32) + bo1_ref[...]
    tb_in = jnp.dot(te, wi1b, preferred_element_type=jnp.float32) + bi1_ref[...]

    xseg = _segment(xt_ref[...])
    yseg = _segment(yt_ref[...])
    tx = jnp.where(xseg == 0, tb_out[0:1], jnp.where(xseg == 1, tb_out[1:2], tb_out[2:3]))
    ty = jnp.where(yseg == 0, tb_in[0:1], jnp.where(yseg == 1, tb_in[1:2], tb_in[2:3]))
    tn = tb_in[0:1]

    h_out = jnp.maximum(jnp.dot(xg, wo1a, preferred_element_type=jnp.float32) + tx, 0.0)
    xi_out = jnp.dot(h_out, wo2_ref[...], preferred_element_type=jnp.float32) + bo2_ref[...]
    h_pos = jnp.maximum(jnp.dot(yg, wi1a, preferred_element_type=jnp.float32) + ty, 0.0)
    xi_pos = jnp.dot(h_pos, wi2_ref[...], preferred_element_type=jnp.float32) + bi2_ref[...]
    h_neg = jnp.maximum(jnp.dot(ng, wi1a, preferred_element_type=jnp.float32) + tn, 0.0)
    xi_neg = jnp.dot(h_neg, wi2_ref[...], preferred_element_type=jnp.float32) + bi2_ref[...]

    dp = xi_out - xi_pos
    dn = xi_out - xi_neg
    pd = jnp.sqrt(jnp.sum(dp * dp, axis=1, keepdims=True))
    nd = jnp.sqrt(jnp.sum(dn * dn, axis=1, keepdims=True))
    z = nd - pd
    ls = jnp.minimum(z, 0.0) - jnp.log1p(jnp.exp(-jnp.abs(z)))
    partial = jnp.sum(ls, keepdims=True).reshape(1, 1)

    @pl.when(i == 0)
    def _init():
        out_ref[...] = jnp.zeros_like(out_ref)

    out_ref[...] += partial

    @pl.when(i == nb - 1)
    def _finish():
        out_ref[...] = out_ref[...] * (-1.0 / n_total)


def _mlp_loss(g, xt, yt, te, wo1, bo1, wo2, bo2, wi1, bi1, wi2, bi2, bt):
    n = g.shape[1]
    d = g.shape[2]
    grid = (n // bt,)
    full = lambda s: pl.BlockSpec(s, lambda i: tuple(0 for _ in s))
    return pl.pallas_call(
        functools.partial(_mlp_loss_body, n),
        grid=grid,
        in_specs=[
            pl.BlockSpec((3, bt, d), lambda i: (0, i, 0)),
            pl.BlockSpec((bt, 1), lambda i: (i, 0)),
            pl.BlockSpec((bt, 1), lambda i: (i, 0)),
            full(te.shape),
            full(wo1.shape), full(bo1.shape), full(wo2.shape), full(bo2.shape),
            full(wi1.shape), full(bi1.shape), full(wi2.shape), full(bi2.shape),
        ],
        out_specs=pl.BlockSpec((1, 1), lambda i: (0, 0)),
        out_shape=jax.ShapeDtypeStruct((1, 1), jnp.float32),
    )(g, xt, yt, te, wo1, bo1, wo2, bo2, wi1, bi1, wi2, bi2)


def kernel(x, x_t_slot, y, y_t_slot, vecs_use, time_embeddings,
           W_out1, b_out1, W_out2, b_out2, W_in1, b_in1, W_in2, b_in2):
    seq_len, user_len = x.shape
    n = seq_len * user_len
    v, d = vecs_use.shape

    neg_idx = jax.random.randint(jax.random.key(1234), (n,), 0, v, dtype=jnp.int32)
    idx_all = jnp.concatenate([x.reshape(-1), y.reshape(-1), neg_idx])

    g = _make_sc_gather(v, d, 3 * n, 4800)(vecs_use, idx_all)
    g = g.reshape(3, n, d)

    loss = _mlp_loss(
        g,
        x_t_slot.reshape(n, 1),
        y_t_slot.reshape(n, 1),
        time_embeddings,
        W_out1, b_out1.reshape(1, d),
        W_out2, b_out2.reshape(1, d),
        W_in1, b_in1.reshape(1, d),
        W_in2, b_in2.reshape(1, d),
        bt=2048,
    )
    return loss.reshape(())
